# baseline (speedup 1.0000x reference)
import functools

import jax
import jax.numpy as jnp
from jax import lax
from jax.experimental import pallas as pl
from jax.experimental.pallas import tpu as pltpu
from jax.experimental.pallas import tpu_sc as plsc

_NS = 16
_HALF = 32

_W0 = _HALF // _NS
_NBUF = 4
_NCHUNK = 64
_C1 = 128 // _NCHUNK


def _make_body(half_idx):
    def _sc_copy(x_hbm, o_hbm, *scratch):
        bufs = scratch[:_NBUF]
        in_sems = scratch[_NBUF:2 * _NBUF]
        out_sems = scratch[2 * _NBUF:]
        wid = lax.axis_index("s")
        src = half_idx * _HALF + wid * _W0
        dst = wid * _W0

        def in_copy(j):
            b = j % _NBUF
            return pltpu.make_async_copy(
                x_hbm.at[pl.ds(src, _W0), pl.ds(j * _C1, _C1)], bufs[b], in_sems[b]
            )

        def out_copy(j):
            b = j % _NBUF
            return pltpu.make_async_copy(
                bufs[b], o_hbm.at[pl.ds(dst, _W0), pl.ds(j * _C1, _C1)], out_sems[b]
            )

        for j in range(min(_NBUF, _NCHUNK)):
            in_copy(j).start()
        for j in range(_NCHUNK):
            in_copy(j).wait()
            out_copy(j).start()
            nxt = j + _NBUF
            if nxt < _NCHUNK:
                out_copy(j).wait()
                in_copy(nxt).start()
        for j in range(max(0, _NCHUNK - _NBUF), _NCHUNK):
            out_copy(j).wait()

    return _sc_copy


def _sc_half(x, half_idx):
    run = functools.partial(
        pl.kernel,
        mesh=plsc.VectorSubcoreMesh(
            core_axis_name="c", subcore_axis_name="s", num_cores=1
        ),
        out_type=jax.ShapeDtypeStruct((_HALF, 128, 56, 56), x.dtype),
        scratch_types=(
            [pltpu.VMEM((_W0, _C1, 56, 56), jnp.float32) for _ in range(_NBUF)]
            + [pltpu.SemaphoreType.DMA for _ in range(2 * _NBUF)]
        ),
    )(_make_body(half_idx))
    return run(x)


def kernel(x):
    a = _sc_half(x, 0)
    b = _sc_half(x, 1)
    return jnp.concatenate([a, b], axis=0)


# speedup vs baseline: 1.0960x; 1.0960x over previous
import functools

import jax
import jax.numpy as jnp
from jax import lax
from jax.experimental import pallas as pl
from jax.experimental.pallas import tpu as pltpu
from jax.experimental.pallas import tpu_sc as plsc

_NC = 2
_NS = 16
_NW = _NC * _NS

_W0 = 64 // _NW
_NBUF = 4
_NCHUNK = 64
_C1 = 128 // _NCHUNK


def _sc_copy(x_hbm, o_hbm, *scratch):
    bufs = scratch[:_NBUF]
    in_sems = scratch[_NBUF:2 * _NBUF]
    out_sems = scratch[2 * _NBUF:]
    wid = lax.axis_index("s") * _NC + lax.axis_index("c")
    base = wid * _W0

    def in_copy(j):
        b = j % _NBUF
        return pltpu.make_async_copy(
            x_hbm.at[pl.ds(base, _W0), pl.ds(j * _C1, _C1)], bufs[b], in_sems[b]
        )

    def out_copy(j):
        b = j % _NBUF
        return pltpu.make_async_copy(
            bufs[b], o_hbm.at[pl.ds(base, _W0), pl.ds(j * _C1, _C1)], out_sems[b]
        )

    for j in range(min(_NBUF, _NCHUNK)):
        in_copy(j).start()
    for j in range(_NCHUNK):
        in_copy(j).wait()
        out_copy(j).start()
        nxt = j + _NBUF
        if nxt < _NCHUNK:
            out_copy(j).wait()
            in_copy(nxt).start()
    for j in range(max(0, _NCHUNK - _NBUF), _NCHUNK):
        out_copy(j).wait()


def kernel(x):
    run = functools.partial(
        pl.kernel,
        mesh=plsc.VectorSubcoreMesh(core_axis_name="c", subcore_axis_name="s"),
        out_type=jax.ShapeDtypeStruct(x.shape, x.dtype),
        scratch_types=(
            [pltpu.VMEM((_W0, _C1, 56, 56), jnp.float32) for _ in range(_NBUF)]
            + [pltpu.SemaphoreType.DMA for _ in range(2 * _NBUF)]
        ),
    )(_sc_copy)
    return run(x)
